# baseline (speedup 1.0000x reference)
import functools

import jax
import jax.numpy as jnp
from jax import lax
from jax.experimental import pallas as pl
from jax.experimental.pallas import tpu as pltpu
from jax.experimental.pallas import tpu_sc as plsc

_N = 1048576
_NC, _NS, _L = 2, 16, 16
_NW = _NC * _NS

_N_SC = 262144
_CHUNK = _N_SC // _NW
_P = _CHUNK // 2
_ALPHA = 0.1

_ROWS = 128
_N_TC = _N - _N_SC
_TC_BLOCK_R = 1024
_TC_OFF_B = _N_SC // _ROWS // _TC_BLOCK_R
_TC_STEPS = _N_TC // _ROWS // _TC_BLOCK_R

_mesh = plsc.VectorSubcoreMesh(core_axis_name="c", subcore_axis_name="s")


@functools.partial(
    pl.kernel,
    mesh=_mesh,
    compiler_params=pltpu.CompilerParams(
        needs_layout_passes=False,
        disable_bounds_checks=True,
        skip_device_barrier=True,
    ),
    out_type=jax.ShapeDtypeStruct((_NW, _L), jnp.float32),
    scratch_types=[
        pltpu.VMEM((2, _P), jnp.float32),
        pltpu.VMEM((2, _P), jnp.int32),
        pltpu.VMEM((100,), jnp.float32),
        pltpu.VMEM((_L,), jnp.float32),
        pltpu.SemaphoreType.DMA,
        pltpu.SemaphoreType.DMA,
        pltpu.SemaphoreType.DMA,
        pltpu.SemaphoreType.DMA,
        pltpu.SemaphoreType.DMA,
    ],
)
def _pinball_sc(s_hbm, y_hbm, t_hbm, out_hbm,
                s_v, y_v, t_v, o_v, sem_s0, sem_s1, sem_y0, sem_y1, sem_t):
    wid = lax.axis_index("s") * _NC + lax.axis_index("c")
    base = wid * _CHUNK
    sem_s = (sem_s0, sem_s1)
    sem_y = (sem_y0, sem_y1)

    cp_t = pltpu.async_copy(t_hbm, t_v, sem_t)

    def start(k):
        b = k % 2
        cs = pltpu.async_copy(
            s_hbm.at[pl.ds(base + k * _P, _P)], s_v.at[b], sem_s[b])
        cy = pltpu.async_copy(
            y_hbm.at[pl.ds(base + k * _P, _P)], y_v.at[b], sem_y[b])
        return cs, cy

    pend = start(0)
    cp_t.wait()

    z = jnp.zeros((_L,), jnp.float32)
    accs = (z, z)
    for k in range(2):
        b = k % 2
        cs, cy = pend
        if k + 1 < 2:
            pend = start(k + 1)
        cs.wait()
        cy.wait()

        def step(i, accs, _b=b):
            a0, a1 = accs
            s0 = s_v[_b, pl.ds(i, _L)]
            idx0 = y_v[_b, pl.ds(i, _L)]
            s1 = s_v[_b, pl.ds(i + _L, _L)]
            idx1 = y_v[_b, pl.ds(i + _L, _L)]
            d0 = plsc.load_gather(t_v, [idx0]) - s0
            d1 = plsc.load_gather(t_v, [idx1]) - s1
            a0 = a0 + jnp.maximum((1.0 - _ALPHA) * d0, -_ALPHA * d0)
            a1 = a1 + jnp.maximum((1.0 - _ALPHA) * d1, -_ALPHA * d1)
            return a0, a1

        accs = plsc.parallel_loop(0, _P, 2 * _L, unroll=8, carry=accs)(step)

    o_v[...] = accs[0] + accs[1]
    pltpu.sync_copy(o_v, out_hbm.at[wid])


def _pinball_tc_body(s_ref, y_ref, t_ref, o_ref, acc_ref, tbl_ref):
    i = pl.program_id(0)

    @pl.when(i == 0)
    def _():
        t128 = jnp.concatenate(
            [t_ref[...], jnp.zeros((_ROWS - 100,), jnp.float32)]
        ).reshape(1, _ROWS)
        tbl_ref[...] = jnp.broadcast_to(t128, (_TC_BLOCK_R, _ROWS))

    q = jnp.take_along_axis(tbl_ref[...], y_ref[...], axis=1,
                            mode="promise_in_bounds")
    d = q - s_ref[...]
    loss = jnp.maximum((1.0 - _ALPHA) * d, -_ALPHA * d)
    part = loss.reshape(_TC_BLOCK_R // 8, 8, _ROWS).sum(axis=0)

    @pl.when(i == 0)
    def _():
        acc_ref[...] = part

    @pl.when(i > 0)
    def _():
        acc_ref[...] += part

    @pl.when(i == _TC_STEPS - 1)
    def _():
        o_ref[0, 0] = jnp.sum(acc_ref[...])


_pinball_tc = pl.pallas_call(
    _pinball_tc_body,
    grid=(_TC_STEPS,),
    in_specs=[
        pl.BlockSpec((_TC_BLOCK_R, _ROWS), lambda i: (i + _TC_OFF_B, 0)),
        pl.BlockSpec((_TC_BLOCK_R, _ROWS), lambda i: (i + _TC_OFF_B, 0)),
        pl.BlockSpec((100,), lambda i: (0,)),
    ],
    out_specs=pl.BlockSpec(memory_space=pltpu.SMEM),
    out_shape=jax.ShapeDtypeStruct((1, 1), jnp.float32),
    scratch_shapes=[pltpu.VMEM((8, _ROWS), jnp.float32),
                    pltpu.VMEM((_TC_BLOCK_R, _ROWS), jnp.float32)],
)


def kernel(S, Y, y_pred):
    Yi = Y.astype(jnp.int32)
    sc_partials = _pinball_sc(S, Yi, y_pred)
    s2 = S.reshape(_N // _ROWS, _ROWS)
    y2 = Yi.reshape(_N // _ROWS, _ROWS)
    tc_part = _pinball_tc(s2, y2, y_pred)
    return (jnp.sum(sc_partials) + tc_part[0, 0]) / _N


# speedup vs baseline: 1.0111x; 1.0111x over previous
import functools

import jax
import jax.numpy as jnp
from jax import lax
from jax.experimental import pallas as pl
from jax.experimental.pallas import tpu as pltpu
from jax.experimental.pallas import tpu_sc as plsc

_N = 1048576
_NC, _NS, _L = 2, 16, 16
_NW = _NC * _NS

_N_SC = 262144
_CHUNK = _N_SC // _NW
_P = _CHUNK // 2
_ALPHA = 0.1

_ROWS = 128
_N_TC = _N - _N_SC
_TC_BLOCK_R = 1024
_TC_OFF_B = _N_SC // _ROWS // _TC_BLOCK_R
_TC_STEPS = _N_TC // _ROWS // _TC_BLOCK_R // 2

_mesh = plsc.VectorSubcoreMesh(core_axis_name="c", subcore_axis_name="s")


@functools.partial(
    pl.kernel,
    mesh=_mesh,
    compiler_params=pltpu.CompilerParams(
        needs_layout_passes=False,
        disable_bounds_checks=True,
        skip_device_barrier=True,
    ),
    out_type=jax.ShapeDtypeStruct((_NW, _L), jnp.float32),
    scratch_types=[
        pltpu.VMEM((2, _P), jnp.float32),
        pltpu.VMEM((2, _P), jnp.int32),
        pltpu.VMEM((100,), jnp.float32),
        pltpu.VMEM((_L,), jnp.float32),
        pltpu.SemaphoreType.DMA,
        pltpu.SemaphoreType.DMA,
        pltpu.SemaphoreType.DMA,
        pltpu.SemaphoreType.DMA,
        pltpu.SemaphoreType.DMA,
    ],
)
def _pinball_sc(s_hbm, y_hbm, t_hbm, out_hbm,
                s_v, y_v, t_v, o_v, sem_s0, sem_s1, sem_y0, sem_y1, sem_t):
    wid = lax.axis_index("s") * _NC + lax.axis_index("c")
    base = wid * _CHUNK
    sem_s = (sem_s0, sem_s1)
    sem_y = (sem_y0, sem_y1)

    cp_t = pltpu.async_copy(t_hbm, t_v, sem_t)

    def start(k):
        b = k % 2
        cs = pltpu.async_copy(
            s_hbm.at[pl.ds(base + k * _P, _P)], s_v.at[b], sem_s[b])
        cy = pltpu.async_copy(
            y_hbm.at[pl.ds(base + k * _P, _P)], y_v.at[b], sem_y[b])
        return cs, cy

    pend = start(0)
    cp_t.wait()

    z = jnp.zeros((_L,), jnp.float32)
    accs = (z, z)
    for k in range(2):
        b = k % 2
        cs, cy = pend
        if k + 1 < 2:
            pend = start(k + 1)
        cs.wait()
        cy.wait()

        def step(i, accs, _b=b):
            a0, a1 = accs
            s0 = s_v[_b, pl.ds(i, _L)]
            idx0 = y_v[_b, pl.ds(i, _L)]
            s1 = s_v[_b, pl.ds(i + _L, _L)]
            idx1 = y_v[_b, pl.ds(i + _L, _L)]
            d0 = plsc.load_gather(t_v, [idx0]) - s0
            d1 = plsc.load_gather(t_v, [idx1]) - s1
            a0 = a0 + jnp.maximum((1.0 - _ALPHA) * d0, -_ALPHA * d0)
            a1 = a1 + jnp.maximum((1.0 - _ALPHA) * d1, -_ALPHA * d1)
            return a0, a1

        accs = plsc.parallel_loop(0, _P, 2 * _L, unroll=8, carry=accs)(step)

    o_v[...] = accs[0] + accs[1]
    pltpu.sync_copy(o_v, out_hbm.at[wid])


def _pinball_tc_body(sa_ref, sb_ref, ya_ref, yb_ref, t_ref, o_ref,
                     acc_ref, tbl_ref):
    i = pl.program_id(0)

    @pl.when(i == 0)
    def _():
        t128 = jnp.concatenate(
            [t_ref[...], jnp.zeros((_ROWS - 100,), jnp.float32)]
        ).reshape(1, _ROWS)
        tbl_ref[...] = jnp.broadcast_to(t128, (_TC_BLOCK_R, _ROWS))

    def blk_loss(s_ref, y_ref):
        q = jnp.take_along_axis(tbl_ref[...], y_ref[...], axis=1,
                                mode="promise_in_bounds")
        d = q - s_ref[...]
        loss = jnp.maximum((1.0 - _ALPHA) * d, -_ALPHA * d)
        return loss.reshape(_TC_BLOCK_R // 8, 8, _ROWS).sum(axis=0)

    part = blk_loss(sa_ref, ya_ref) + blk_loss(sb_ref, yb_ref)

    @pl.when(i == 0)
    def _():
        acc_ref[...] = part

    @pl.when(i > 0)
    def _():
        acc_ref[...] += part

    @pl.when(i == _TC_STEPS - 1)
    def _():
        o_ref[0, 0] = jnp.sum(acc_ref[...])


_pinball_tc = pl.pallas_call(
    _pinball_tc_body,
    grid=(_TC_STEPS,),
    in_specs=[
        pl.BlockSpec((_TC_BLOCK_R, _ROWS), lambda i: (i + _TC_OFF_B, 0)),
        pl.BlockSpec((_TC_BLOCK_R, _ROWS),
                     lambda i: (i + _TC_OFF_B + _TC_STEPS, 0)),
        pl.BlockSpec((_TC_BLOCK_R, _ROWS), lambda i: (i + _TC_OFF_B, 0)),
        pl.BlockSpec((_TC_BLOCK_R, _ROWS),
                     lambda i: (i + _TC_OFF_B + _TC_STEPS, 0)),
        pl.BlockSpec((100,), lambda i: (0,)),
    ],
    out_specs=pl.BlockSpec(memory_space=pltpu.SMEM),
    out_shape=jax.ShapeDtypeStruct((1, 1), jnp.float32),
    scratch_shapes=[pltpu.VMEM((8, _ROWS), jnp.float32),
                    pltpu.VMEM((_TC_BLOCK_R, _ROWS), jnp.float32)],
)


def kernel(S, Y, y_pred):
    Yi = Y.astype(jnp.int32)
    sc_partials = _pinball_sc(S, Yi, y_pred)
    s2 = S.reshape(_N // _ROWS, _ROWS)
    y2 = Yi.reshape(_N // _ROWS, _ROWS)
    tc_part = _pinball_tc(s2, s2, y2, y2, y_pred)
    return (jnp.sum(sc_partials) + tc_part[0, 0]) / _N
